# baseline (speedup 1.0000x reference)
import functools

import jax
import jax.numpy as jnp
from jax import lax
from jax.experimental import pallas as pl
from jax.experimental.pallas import tpu as pltpu
from jax.experimental.pallas import tpu_sc as plsc

B = 16384
D = 16
NUM_CORES = 2
NUM_SUBCORES = 16
NW = NUM_CORES * NUM_SUBCORES
BPW = B // NW

_mesh = plsc.VectorSubcoreMesh(core_axis_name="c", subcore_axis_name="s")


@functools.partial(
    pl.kernel,
    mesh=_mesh,
    out_type=jax.ShapeDtypeStruct((B,), jnp.float32),
    scratch_types=[
        pltpu.VMEM((BPW,), jnp.int32),
        pltpu.VMEM((BPW,), jnp.int32),
        pltpu.VMEM((BPW, D), jnp.float32),
        pltpu.VMEM((BPW, D), jnp.float32),
        pltpu.VMEM((BPW,), jnp.float32),
        pltpu.VMEM((16 * 17,), jnp.float32),
        pltpu.SemaphoreType.DMA,
        pltpu.SemaphoreType.DMA,
    ],
    compiler_params=pltpu.CompilerParams(
        needs_layout_passes=False, use_tc_tiling_on_sc=False),
)
def _mf_kernel(uidx_hbm, iidx_hbm, utab_hbm, itab_hbm, out_hbm,
               uidx_v, iidx_v, urows_v, irows_v, res_v, pt_v, sem_u, sem_i):
    wid = lax.axis_index("s") * NUM_CORES + lax.axis_index("c")
    base = wid * BPW
    pltpu.sync_copy(uidx_hbm.at[pl.ds(base, BPW)], uidx_v)
    pltpu.sync_copy(iidx_hbm.at[pl.ds(base, BPW)], iidx_v)
    cu = pltpu.async_copy(utab_hbm.at[uidx_v], urows_v, sem_u)
    ci = pltpu.async_copy(itab_hbm.at[iidx_v], irows_v, sem_i)
    cu.wait()
    ci.wait()

    lane = lax.iota(jnp.int32, 16)

    def body(g, _):
        row0 = g * 16
        for j in range(16):
            pt_v[pl.ds(j * 17, 16)] = urows_v[row0 + j, :] * irows_v[row0 + j, :]
        row_addr = lane * 17
        acc = jnp.zeros((16,), jnp.float32)
        for d in range(16):
            acc = acc + plsc.load_gather(pt_v, [row_addr + d])
        res_v[pl.ds(row0, 16)] = acc
        return _

    lax.fori_loop(0, BPW // 16, body, 0, unroll=2)

    pltpu.sync_copy(res_v, out_hbm.at[pl.ds(base, BPW)])


def kernel(user_input, item_input, user_table, item_table):
    out = _mf_kernel(user_input.astype(jnp.int32),
                     item_input.astype(jnp.int32),
                     user_table, item_table)
    return out.reshape(B, 1)


# speedup vs baseline: 5.0729x; 5.0729x over previous
import functools

import jax
import jax.numpy as jnp
from jax import lax
from jax.experimental import pallas as pl
from jax.experimental.pallas import tpu as pltpu
from jax.experimental.pallas import tpu_sc as plsc

B = 16384
D = 16
NUM_CORES = 2
NUM_SUBCORES = 16
NW = NUM_CORES * NUM_SUBCORES
BPW = B // NW
G = 16
NGRP = BPW // G
WPAD = 130

_mesh = plsc.VectorSubcoreMesh(core_axis_name="c", subcore_axis_name="s")


@functools.partial(
    pl.kernel,
    mesh=_mesh,
    out_type=jax.ShapeDtypeStruct((B,), jnp.float32),
    scratch_types=[
        pltpu.VMEM((BPW,), jnp.int32),
        pltpu.VMEM((BPW,), jnp.int32),
        pltpu.VMEM((4, D, WPAD), jnp.float32),
        pltpu.VMEM((4, D, WPAD), jnp.float32),
        pltpu.VMEM((16 * 17,), jnp.float32),
        pltpu.VMEM((BPW,), jnp.float32),
        pltpu.SemaphoreType.DMA,
        pltpu.SemaphoreType.DMA,
    ],
    compiler_params=pltpu.CompilerParams(needs_layout_passes=False),
)
def _mf_kernel(uidx_hbm, iidx_hbm, utab_hbm, itab_hbm, out_hbm,
               uidx_v, iidx_v, uw_v, iw_v, pt_v, res_v, sem_u, sem_i):
    wid = lax.axis_index("s") * NUM_CORES + lax.axis_index("c")
    base = wid * BPW
    pltpu.sync_copy(uidx_hbm.at[pl.ds(base, BPW)], uidx_v)
    pltpu.sync_copy(iidx_hbm.at[pl.ds(base, BPW)], iidx_v)

    lane = lax.iota(jnp.int32, 16)

    def issue(ru, ri, j, slot):
        cu = pl.multiple_of(lax.shift_right_logical(ru[j], 7) * 128, 128)
        ci = pl.multiple_of(lax.shift_right_logical(ri[j], 7) * 128, 128)
        pltpu.async_copy(utab_hbm.at[:, pl.ds(cu, 128)],
                         uw_v.at[slot, :, pl.ds(0, 128)], sem_u)
        pltpu.async_copy(itab_hbm.at[:, pl.ds(ci, 128)],
                         iw_v.at[slot, :, pl.ds(0, 128)], sem_i)

    def body(g, carry):
        ru = uidx_v[pl.ds(g * G, G)]
        ri = iidx_v[pl.ds(g * G, G)]
        for j in range(4):
            issue(ru, ri, j, j)
        for j in range(G):
            slot = j % 4
            pltpu.make_async_copy(utab_hbm.at[:, pl.ds(0, 128)],
                                  uw_v.at[slot, :, pl.ds(0, 128)], sem_u).wait()
            pltpu.make_async_copy(itab_hbm.at[:, pl.ds(0, 128)],
                                  iw_v.at[slot, :, pl.ds(0, 128)], sem_i).wait()
            svec = jnp.full((16,), slot, dtype=jnp.int32)
            lu = jnp.full((16,), 0, dtype=jnp.int32) + lax.bitwise_and(ru[j], 127)
            li = jnp.full((16,), 0, dtype=jnp.int32) + lax.bitwise_and(ri[j], 127)
            uvec = plsc.load_gather(uw_v, [svec, lane, lu])
            ivec = plsc.load_gather(iw_v, [svec, lane, li])
            pt_v[pl.ds(j * 17, 16)] = uvec * ivec
            if j + 4 < G:
                issue(ru, ri, j + 4, slot)
        row_addr = lane * 17
        acc = jnp.zeros((16,), jnp.float32)
        for d in range(16):
            acc = acc + plsc.load_gather(pt_v, [row_addr + d])
        res_v[pl.ds(g * G, G)] = acc
        return carry

    lax.fori_loop(0, NGRP, body, 0)

    pltpu.sync_copy(res_v, out_hbm.at[pl.ds(base, BPW)])


def kernel(user_input, item_input, user_table, item_table):
    out = _mf_kernel(user_input.astype(jnp.int32),
                     item_input.astype(jnp.int32),
                     user_table.T, item_table.T)
    return out.reshape(B, 1)


# speedup vs baseline: 6.0175x; 1.1862x over previous
import functools

import jax
import jax.numpy as jnp
from jax import lax
from jax.experimental import pallas as pl
from jax.experimental.pallas import tpu as pltpu
from jax.experimental.pallas import tpu_sc as plsc

B = 16384
D = 16
NUM_CORES = 2
NUM_SUBCORES = 16
NW = NUM_CORES * NUM_SUBCORES
BPW = B // NW
G = 16
NGRP = BPW // G
WPAD = 130

_mesh = plsc.VectorSubcoreMesh(core_axis_name="c", subcore_axis_name="s")


@functools.partial(
    pl.kernel,
    mesh=_mesh,
    out_type=jax.ShapeDtypeStruct((B,), jnp.float32),
    scratch_types=[
        pltpu.VMEM((BPW,), jnp.int32),
        pltpu.VMEM((BPW,), jnp.int32),
        pltpu.VMEM((8, D, WPAD), jnp.float32),
        pltpu.VMEM((8, D, WPAD), jnp.float32),
        pltpu.VMEM((16 * 17,), jnp.float32),
        pltpu.VMEM((BPW,), jnp.float32),
        pltpu.SemaphoreType.DMA,
        pltpu.SemaphoreType.DMA,
    ],
    compiler_params=pltpu.CompilerParams(needs_layout_passes=False),
)
def _mf_kernel(uidx_hbm, iidx_hbm, utab_hbm, itab_hbm, out_hbm,
               uidx_v, iidx_v, uw_v, iw_v, pt_v, res_v, sem_u, sem_i):
    wid = lax.axis_index("s") * NUM_CORES + lax.axis_index("c")
    base = wid * BPW
    pltpu.sync_copy(uidx_hbm.at[pl.ds(base, BPW)], uidx_v)
    pltpu.sync_copy(iidx_hbm.at[pl.ds(base, BPW)], iidx_v)

    lane = lax.iota(jnp.int32, 16)

    def issue(ru, ri, j, slot):
        cu = pl.multiple_of(lax.shift_right_logical(ru[j], 7) * 128, 128)
        ci = pl.multiple_of(lax.shift_right_logical(ri[j], 7) * 128, 128)
        pltpu.async_copy(utab_hbm.at[:, pl.ds(cu, 128)],
                         uw_v.at[slot, :, pl.ds(0, 128)], sem_u)
        pltpu.async_copy(itab_hbm.at[:, pl.ds(ci, 128)],
                         iw_v.at[slot, :, pl.ds(0, 128)], sem_i)

    def body(g, carry):
        ru = uidx_v[pl.ds(g * G, G)]
        ri = iidx_v[pl.ds(g * G, G)]
        for j in range(8):
            issue(ru, ri, j, j)
        for j in range(G):
            slot = j % 8
            pltpu.make_async_copy(utab_hbm.at[:, pl.ds(0, 128)],
                                  uw_v.at[slot, :, pl.ds(0, 128)], sem_u).wait()
            pltpu.make_async_copy(itab_hbm.at[:, pl.ds(0, 128)],
                                  iw_v.at[slot, :, pl.ds(0, 128)], sem_i).wait()
            svec = jnp.full((16,), slot, dtype=jnp.int32)
            lu = jnp.full((16,), 0, dtype=jnp.int32) + lax.bitwise_and(ru[j], 127)
            li = jnp.full((16,), 0, dtype=jnp.int32) + lax.bitwise_and(ri[j], 127)
            uvec = plsc.load_gather(uw_v, [svec, lane, lu])
            ivec = plsc.load_gather(iw_v, [svec, lane, li])
            pt_v[pl.ds(j * 17, 16)] = uvec * ivec
            if j + 8 < G:
                issue(ru, ri, j + 8, slot)
        row_addr = lane * 17
        acc = jnp.zeros((16,), jnp.float32)
        for d in range(16):
            acc = acc + plsc.load_gather(pt_v, [row_addr + d])
        res_v[pl.ds(g * G, G)] = acc
        return carry

    lax.fori_loop(0, NGRP, body, 0)

    pltpu.sync_copy(res_v, out_hbm.at[pl.ds(base, BPW)])


def kernel(user_input, item_input, user_table, item_table):
    out = _mf_kernel(user_input.astype(jnp.int32),
                     item_input.astype(jnp.int32),
                     user_table.T, item_table.T)
    return out.reshape(B, 1)
